# Initial kernel scaffold; baseline (speedup 1.0000x reference)
import jax
import jax.numpy as jnp
from jax.experimental import pallas as pl


def kernel(x_user, x_movie, edge_index_um, edge_index_mu, Wl0_um, bl0_um, Wr0_um, Wl0_mu, bl0_mu, Wr0_mu, Wl1_um, bl1_um, Wr1_um, Wl1_mu, bl1_mu, Wr1_mu):
    raise NotImplementedError("write your pallas kernel here")


# speedup vs baseline: 3.5408x; 3.5408x over previous
import functools

import jax
import jax.numpy as jnp
from jax import lax
from jax.experimental import pallas as pl
from jax.experimental.pallas import tpu as pltpu
from jax.experimental.pallas import tpu_sc as plsc

N = 10000
E = 160000
D_IN = 256
D_HID = 256
D_OUT = 128

NC = 2
NS = 16
G = 128
NCHUNK = E // G
NP_ = 10240
RPT = NP_ // NS

_mesh = plsc.VectorSubcoreMesh(core_axis_name="c", subcore_axis_name="s")


@functools.partial(
    pl.kernel,
    mesh=_mesh,
    out_type=(
        jax.ShapeDtypeStruct((NC, NP_, 128), jnp.float32),
        jax.ShapeDtypeStruct((NC, NP_, 128), jnp.float32),
    ),
    scratch_types=(
        pltpu.VMEM_SHARED((NP_, 128), jnp.float32),
        pltpu.VMEM((G,), jnp.int32),
        pltpu.VMEM((G,), jnp.int32),
        pltpu.VMEM((G, 128), jnp.float32),
        pltpu.SemaphoreType.DMA,
    ),
)
def _agg0(xu2, xm2, s_um, d_um, s_mu, d_mu, ones_hbm, zf_hbm, zc_hbm,
          aggm, aggu, acc, idxs, idxd, rows, sem):
    cid = lax.axis_index("c")
    sid = lax.axis_index("s")
    nj = jnp.where(sid < NCHUNK % NS, NCHUNK // NS + 1, NCHUNK // NS)
    off = cid * N
    nz = RPT // G

    def one_pass(x_hbm, s_hbm, d_hbm, agg_out, cnt_out):
        pltpu.sync_copy(zf_hbm, rows)

        def zbody(z, carry):
            ch = pl.ds(sid * RPT + z * G, G)
            pltpu.sync_copy(rows, acc.at[ch])
            return carry

        lax.fori_loop(0, nz, zbody, 0)
        plsc.subcore_barrier()

        def body(j, carry):
            base = (sid + j * NS) * G
            pltpu.sync_copy(s_hbm.at[pl.ds(base, G)], idxs)
            pltpu.sync_copy(d_hbm.at[pl.ds(base, G)], idxd)
            for k in range(G // 16):
                idxs[pl.ds(k * 16, 16)] = idxs[pl.ds(k * 16, 16)] + off
            pltpu.async_copy(x_hbm.at[idxs], rows, sem).wait()
            pltpu.sync_copy(rows, acc.at[idxd], add=True)
            return carry

        lax.fori_loop(0, nj, body, 0)
        plsc.subcore_barrier()

        def wbody(z, carry):
            ch = pl.ds(sid * RPT + z * G, G)
            pltpu.sync_copy(acc.at[ch], rows)
            pltpu.sync_copy(rows, agg_out.at[cid].at[ch])
            return carry

        lax.fori_loop(0, nz, wbody, 0)

    one_pass(xu2, s_um, d_um, aggm, None)
    one_pass(xm2, s_mu, d_mu, aggu, None)


@functools.partial(
    pl.kernel,
    mesh=_mesh,
    compiler_params=pltpu.CompilerParams(needs_layout_passes=False),
    out_type=(
        jax.ShapeDtypeStruct((NC, NS, NP_ // 16, 16), jnp.float32),
        jax.ShapeDtypeStruct((NC, NS, NP_ // 16, 16), jnp.float32),
    ),
    scratch_types=(
        pltpu.VMEM((G,), jnp.int32),
        pltpu.VMEM((NP_ // 16, 16), jnp.float32),
    ),
)
def _degrees(d_um, d_mu, zcnt_hbm, cm_out, cu_out, idxd, c_v):
    cid = lax.axis_index("c")
    sid = lax.axis_index("s")
    wid = cid * NS + sid
    nw = NC * NS
    nj = jnp.where(wid < NCHUNK % nw, NCHUNK // nw + 1, NCHUNK // nw)
    ones16 = jnp.ones((16,), jnp.float32)

    def one(d_hbm, out):
        pltpu.sync_copy(zcnt_hbm, c_v)

        def body(j, carry):
            base = (wid + j * nw) * G
            pltpu.sync_copy(d_hbm.at[pl.ds(base, G)], idxd)
            for k in range(G // 16):
                iv = idxd[pl.ds(k * 16, 16)]
                plsc.addupdate_scatter(
                    c_v,
                    [lax.shift_right_logical(iv, 4), lax.bitwise_and(iv, 15)],
                    ones16)
            return carry

        lax.fori_loop(0, nj, body, 0)
        pltpu.sync_copy(c_v, out.at[cid].at[sid])

    one(d_um, cm_out)
    one(d_mu, cu_out)


@functools.partial(
    pl.kernel,
    mesh=_mesh,
    out_type=(
        jax.ShapeDtypeStruct((NC, NP_, 128), jnp.float32),
        jax.ShapeDtypeStruct((NC, NP_, 128), jnp.float32),
    ),
    scratch_types=(
        pltpu.VMEM_SHARED((NP_, 128), jnp.float32),
        pltpu.VMEM((G,), jnp.int32),
        pltpu.VMEM((G,), jnp.int32),
        pltpu.VMEM((G, 128), jnp.float32),
        pltpu.SemaphoreType.DMA,
    ),
)
def _agg1(tu, tm, s_um, d_um, s_mu, d_mu, zf_hbm,
          agg1m, agg1u, acc, idxs, idxd, rows, sem):
    cid = lax.axis_index("c")
    sid = lax.axis_index("s")
    half = NCHUNK // NC
    nj = jnp.where(sid < half % NS, half // NS + 1, half // NS)

    def one_pass(x_hbm, s_hbm, d_hbm, out_slot):
        pltpu.sync_copy(zf_hbm, rows)

        def zbody(z, carry):
            pltpu.sync_copy(rows, acc.at[pl.ds(sid * RPT + z * G, G)])
            return carry

        lax.fori_loop(0, RPT // G, zbody, 0)
        plsc.subcore_barrier()

        def body(j, carry):
            base = (cid * half + sid + j * NS) * G
            pltpu.sync_copy(s_hbm.at[pl.ds(base, G)], idxs)
            pltpu.sync_copy(d_hbm.at[pl.ds(base, G)], idxd)
            pltpu.async_copy(x_hbm.at[idxs], rows, sem).wait()
            pltpu.sync_copy(rows, acc.at[idxd], add=True)
            return carry

        lax.fori_loop(0, nj, body, 0)
        plsc.subcore_barrier()

        def wbody(z, carry):
            ch = pl.ds(sid * RPT + z * G, G)
            pltpu.sync_copy(acc.at[ch], rows)
            pltpu.sync_copy(rows, out_slot.at[ch])
            return carry

        lax.fori_loop(0, RPT // G, wbody, 0)

    one_pass(tu, s_um, d_um, agg1m.at[cid])
    one_pass(tm, s_mu, d_mu, agg1u.at[cid])


_BM = 1024


def _dense0_body(agg_ref, cnt_ref, x_ref, wl0_ref, bl0_ref, wr0_ref,
                 wl1_ref, wr1_ref, b1_ref, t_ref, p_ref):
    i = pl.program_id(0)
    c = jnp.sum(cnt_ref[:, pl.ds(i * _BM, _BM)], axis=0)[:, None]
    inv = 1.0 / jnp.maximum(c, 1.0)
    m0 = agg_ref[0] * inv
    m1 = agg_ref[1] * inv
    h = (jnp.dot(m0, wl0_ref[0:128, :], preferred_element_type=jnp.float32)
         + jnp.dot(m1, wl0_ref[128:256, :], preferred_element_type=jnp.float32)
         + jnp.dot(x_ref[...], wr0_ref[...], preferred_element_type=jnp.float32)
         + bl0_ref[...])
    h = jnp.maximum(h, 0.0)
    t_ref[...] = jnp.dot(h, wl1_ref[...], preferred_element_type=jnp.float32)
    p_ref[...] = jnp.dot(h, wr1_ref[...], preferred_element_type=jnp.float32) + b1_ref[...]


def _dense0(agg, cnt, x, wl0, bl0, wr0, wl1t, wr1p, b1p):
    grid = (NP_ // _BM,)
    return pl.pallas_call(
        _dense0_body,
        grid=grid,
        in_specs=[
            pl.BlockSpec((NC, _BM, 128), lambda i: (0, i, 0)),
            pl.BlockSpec((NC * NS, NP_), lambda i: (0, 0)),
            pl.BlockSpec((_BM, D_IN), lambda i: (i, 0)),
            pl.BlockSpec((D_IN, D_HID), lambda i: (0, 0)),
            pl.BlockSpec((1, D_HID), lambda i: (0, 0)),
            pl.BlockSpec((D_IN, D_HID), lambda i: (0, 0)),
            pl.BlockSpec((D_HID, D_OUT), lambda i: (0, 0)),
            pl.BlockSpec((D_HID, D_OUT), lambda i: (0, 0)),
            pl.BlockSpec((1, D_OUT), lambda i: (0, 0)),
        ],
        out_specs=[
            pl.BlockSpec((_BM, D_OUT), lambda i: (i, 0)),
            pl.BlockSpec((_BM, D_OUT), lambda i: (i, 0)),
        ],
        out_shape=[
            jax.ShapeDtypeStruct((NP_, D_OUT), jnp.float32),
            jax.ShapeDtypeStruct((NP_, D_OUT), jnp.float32),
        ],
    )(agg, cnt, x, wl0, bl0, wr0, wl1t, wr1p, b1p)


def _combine_body(a_ref, c_ref, p_ref, o_ref):
    i = pl.program_id(0)
    c = jnp.sum(c_ref[:, pl.ds(i * _BM, _BM)], axis=0)[:, None]
    inv = 1.0 / jnp.maximum(c, 1.0)
    o_ref[...] = (a_ref[0] + a_ref[1]) * inv + p_ref[...]


def _combine(agg1, cnt, p):
    grid = (NP_ // _BM,)
    return pl.pallas_call(
        _combine_body,
        grid=grid,
        in_specs=[
            pl.BlockSpec((NC, _BM, D_OUT), lambda i: (0, i, 0)),
            pl.BlockSpec((NC * NS, NP_), lambda i: (0, 0)),
            pl.BlockSpec((_BM, D_OUT), lambda i: (i, 0)),
        ],
        out_specs=pl.BlockSpec((_BM, D_OUT), lambda i: (i, 0)),
        out_shape=jax.ShapeDtypeStruct((NP_, D_OUT), jnp.float32),
    )(agg1, cnt, p)


def kernel(x_user, x_movie, edge_index_um, edge_index_mu,
           Wl0_um, bl0_um, Wr0_um, Wl0_mu, bl0_mu, Wr0_mu,
           Wl1_um, bl1_um, Wr1_um, Wl1_mu, bl1_mu, Wr1_mu):
    f32 = jnp.float32
    s_um = edge_index_um[0].astype(jnp.int32)
    d_um = edge_index_um[1].astype(jnp.int32)
    s_mu = edge_index_mu[0].astype(jnp.int32)
    d_mu = edge_index_mu[1].astype(jnp.int32)
    xu2 = jnp.concatenate([x_user[:, :128], x_user[:, 128:]], axis=0)
    xm2 = jnp.concatenate([x_movie[:, :128], x_movie[:, 128:]], axis=0)
    ones_hbm = jnp.ones((G, 16), f32)
    zf = jnp.zeros((G, 128), f32)
    zc = jnp.zeros((G, 16), f32)

    aggm, aggu = _agg0(xu2, xm2, s_um, d_um, s_mu, d_mu,
                       ones_hbm, zf, zc)
    zcnt = jnp.zeros((NP_ // 16, 16), f32)
    cm4, cu4 = _degrees(d_um, d_mu, zcnt)
    cntm = cm4.reshape(NC * NS, NP_)
    cntu = cu4.reshape(NC * NS, NP_)

    xmp = jnp.pad(x_movie, ((0, NP_ - N), (0, 0)))
    xup = jnp.pad(x_user, ((0, NP_ - N), (0, 0)))
    t_m, p_m = _dense0(aggm, cntm, xmp, Wl0_um, bl0_um.reshape(1, -1), Wr0_um,
                       Wl1_mu, Wr1_um, bl1_um.reshape(1, -1))
    t_u, p_u = _dense0(aggu, cntu, xup, Wl0_mu, bl0_mu.reshape(1, -1), Wr0_mu,
                       Wl1_um, Wr1_mu, bl1_mu.reshape(1, -1))

    agg1m, agg1u = _agg1(t_u, t_m, s_um, d_um, s_mu, d_mu, zf)

    out_movie = _combine(agg1m, cntm, p_m)[:N]
    out_user = _combine(agg1u, cntu, p_u)[:N]
    return (out_user, out_movie)


# speedup vs baseline: 3.6577x; 1.0330x over previous
import functools

import jax
import jax.numpy as jnp
from jax import lax
from jax.experimental import pallas as pl
from jax.experimental.pallas import tpu as pltpu
from jax.experimental.pallas import tpu_sc as plsc

N = 10000
E = 160000
D_IN = 256
D_HID = 256
D_OUT = 128

NC = 2
NS = 16
G = 128
NCHUNK = E // G
NP_ = 10240
RPT = NP_ // NS

_mesh = plsc.VectorSubcoreMesh(core_axis_name="c", subcore_axis_name="s")


@functools.partial(
    pl.kernel,
    mesh=_mesh,
    out_type=(
        jax.ShapeDtypeStruct((NC, NP_, 128), jnp.float32),
        jax.ShapeDtypeStruct((NC, NP_, 128), jnp.float32),
    ),
    scratch_types=(
        pltpu.VMEM_SHARED((NP_, 128), jnp.float32),
        pltpu.VMEM((G,), jnp.int32),
        pltpu.VMEM((G,), jnp.int32),
        pltpu.VMEM((G, 128), jnp.float32),
        pltpu.SemaphoreType.DMA,
    ),
)
def _agg0(xu2, xm2, s_um, d_um, s_mu, d_mu, ones_hbm, zf_hbm, zc_hbm,
          aggm, aggu, acc, idxs, idxd, rows, sem):
    cid = lax.axis_index("c")
    sid = lax.axis_index("s")
    nj = jnp.where(sid < NCHUNK % NS, NCHUNK // NS + 1, NCHUNK // NS)
    off = cid * N
    nz = RPT // G

    def one_pass(x_hbm, s_hbm, d_hbm, agg_out, cnt_out):
        pltpu.sync_copy(zf_hbm, rows)

        def zbody(z, carry):
            ch = pl.ds(sid * RPT + z * G, G)
            pltpu.sync_copy(rows, acc.at[ch])
            return carry

        lax.fori_loop(0, nz, zbody, 0)
        plsc.subcore_barrier()

        def body(j, carry):
            base = (sid + j * NS) * G
            pltpu.sync_copy(s_hbm.at[pl.ds(base, G)], idxs)
            pltpu.sync_copy(d_hbm.at[pl.ds(base, G)], idxd)
            for k in range(G // 16):
                idxs[pl.ds(k * 16, 16)] = idxs[pl.ds(k * 16, 16)] + off
            pltpu.async_copy(x_hbm.at[idxs], rows, sem).wait()
            pltpu.sync_copy(rows, acc.at[idxd], add=True)
            return carry

        lax.fori_loop(0, nj, body, 0)
        plsc.subcore_barrier()

        def wbody(z, carry):
            ch = pl.ds(sid * RPT + z * G, G)
            pltpu.sync_copy(acc.at[ch], rows)
            pltpu.sync_copy(rows, agg_out.at[cid].at[ch])
            return carry

        lax.fori_loop(0, nz, wbody, 0)

    one_pass(xu2, s_um, d_um, aggm, None)
    one_pass(xm2, s_mu, d_mu, aggu, None)


@functools.partial(
    pl.kernel,
    mesh=_mesh,
    compiler_params=pltpu.CompilerParams(needs_layout_passes=False),
    out_type=(
        jax.ShapeDtypeStruct((NC, NS, NP_ // 16, 16), jnp.float32),
        jax.ShapeDtypeStruct((NC, NS, NP_ // 16, 16), jnp.float32),
    ),
    scratch_types=(
        pltpu.VMEM((8 * G,), jnp.int32),
        pltpu.VMEM((NP_ // 16, 16), jnp.float32),
    ),
)
def _degrees(d_um, d_mu, zcnt_hbm, cm_out, cu_out, idxd, c_v):
    cid = lax.axis_index("c")
    sid = lax.axis_index("s")
    wid = cid * NS + sid
    nw = NC * NS
    ndeg = 1280 // nw
    B = 8
    ones16 = jnp.ones((16,), jnp.float32)

    def one(d_hbm, out):
        pltpu.sync_copy(zcnt_hbm, c_v)

        def body(jb, carry):
            base = (wid * ndeg + jb * B) * G
            pltpu.sync_copy(d_hbm.at[pl.ds(base, B * G)], idxd)
            for k in range(B * G // 16):
                iv = idxd[pl.ds(k * 16, 16)]
                plsc.addupdate_scatter(
                    c_v,
                    [lax.shift_right_logical(iv, 4), lax.bitwise_and(iv, 15)],
                    ones16)
            return carry

        lax.fori_loop(0, ndeg // B, body, 0)
        pltpu.sync_copy(c_v, out.at[cid].at[sid])

    one(d_um, cm_out)
    one(d_mu, cu_out)


@functools.partial(
    pl.kernel,
    mesh=_mesh,
    out_type=(
        jax.ShapeDtypeStruct((NC, NP_, 128), jnp.float32),
        jax.ShapeDtypeStruct((NC, NP_, 128), jnp.float32),
    ),
    scratch_types=(
        pltpu.VMEM_SHARED((NP_, 128), jnp.float32),
        pltpu.VMEM((G,), jnp.int32),
        pltpu.VMEM((G,), jnp.int32),
        pltpu.VMEM((G, 128), jnp.float32),
        pltpu.SemaphoreType.DMA,
    ),
)
def _agg1(tu, tm, s_um, d_um, s_mu, d_mu, zf_hbm,
          agg1m, agg1u, acc, idxs, idxd, rows, sem):
    cid = lax.axis_index("c")
    sid = lax.axis_index("s")
    half = NCHUNK // NC
    nj = jnp.where(sid < half % NS, half // NS + 1, half // NS)

    def one_pass(x_hbm, s_hbm, d_hbm, out_slot):
        pltpu.sync_copy(zf_hbm, rows)

        def zbody(z, carry):
            pltpu.sync_copy(rows, acc.at[pl.ds(sid * RPT + z * G, G)])
            return carry

        lax.fori_loop(0, RPT // G, zbody, 0)
        plsc.subcore_barrier()

        def body(j, carry):
            base = (cid * half + sid + j * NS) * G
            pltpu.sync_copy(s_hbm.at[pl.ds(base, G)], idxs)
            pltpu.sync_copy(d_hbm.at[pl.ds(base, G)], idxd)
            pltpu.async_copy(x_hbm.at[idxs], rows, sem).wait()
            pltpu.sync_copy(rows, acc.at[idxd], add=True)
            return carry

        lax.fori_loop(0, nj, body, 0)
        plsc.subcore_barrier()

        def wbody(z, carry):
            ch = pl.ds(sid * RPT + z * G, G)
            pltpu.sync_copy(acc.at[ch], rows)
            pltpu.sync_copy(rows, out_slot.at[ch])
            return carry

        lax.fori_loop(0, RPT // G, wbody, 0)

    one_pass(tu, s_um, d_um, agg1m.at[cid])
    one_pass(tm, s_mu, d_mu, agg1u.at[cid])


_BM = 1024


def _dense0_body(agg_ref, cnt_ref, x_ref, wl0_ref, bl0_ref, wr0_ref,
                 wl1_ref, wr1_ref, b1_ref, t_ref, p_ref):
    i = pl.program_id(0)
    c = jnp.sum(cnt_ref[:, pl.ds(i * _BM, _BM)], axis=0)[:, None]
    inv = 1.0 / jnp.maximum(c, 1.0)
    m0 = agg_ref[0] * inv
    m1 = agg_ref[1] * inv
    h = (jnp.dot(m0, wl0_ref[0:128, :], preferred_element_type=jnp.float32)
         + jnp.dot(m1, wl0_ref[128:256, :], preferred_element_type=jnp.float32)
         + jnp.dot(x_ref[...], wr0_ref[...], preferred_element_type=jnp.float32)
         + bl0_ref[...])
    h = jnp.maximum(h, 0.0)
    t_ref[...] = jnp.dot(h, wl1_ref[...], preferred_element_type=jnp.float32)
    p_ref[...] = jnp.dot(h, wr1_ref[...], preferred_element_type=jnp.float32) + b1_ref[...]


def _dense0(agg, cnt, x, wl0, bl0, wr0, wl1t, wr1p, b1p):
    grid = (NP_ // _BM,)
    return pl.pallas_call(
        _dense0_body,
        grid=grid,
        in_specs=[
            pl.BlockSpec((NC, _BM, 128), lambda i: (0, i, 0)),
            pl.BlockSpec((NC * NS, NP_), lambda i: (0, 0)),
            pl.BlockSpec((_BM, D_IN), lambda i: (i, 0)),
            pl.BlockSpec((D_IN, D_HID), lambda i: (0, 0)),
            pl.BlockSpec((1, D_HID), lambda i: (0, 0)),
            pl.BlockSpec((D_IN, D_HID), lambda i: (0, 0)),
            pl.BlockSpec((D_HID, D_OUT), lambda i: (0, 0)),
            pl.BlockSpec((D_HID, D_OUT), lambda i: (0, 0)),
            pl.BlockSpec((1, D_OUT), lambda i: (0, 0)),
        ],
        out_specs=[
            pl.BlockSpec((_BM, D_OUT), lambda i: (i, 0)),
            pl.BlockSpec((_BM, D_OUT), lambda i: (i, 0)),
        ],
        out_shape=[
            jax.ShapeDtypeStruct((NP_, D_OUT), jnp.float32),
            jax.ShapeDtypeStruct((NP_, D_OUT), jnp.float32),
        ],
    )(agg, cnt, x, wl0, bl0, wr0, wl1t, wr1p, b1p)


def _combine_body(a_ref, c_ref, p_ref, o_ref):
    i = pl.program_id(0)
    c = jnp.sum(c_ref[:, pl.ds(i * _BM, _BM)], axis=0)[:, None]
    inv = 1.0 / jnp.maximum(c, 1.0)
    o_ref[...] = (a_ref[0] + a_ref[1]) * inv + p_ref[...]


def _combine(agg1, cnt, p):
    grid = (NP_ // _BM,)
    return pl.pallas_call(
        _combine_body,
        grid=grid,
        in_specs=[
            pl.BlockSpec((NC, _BM, D_OUT), lambda i: (0, i, 0)),
            pl.BlockSpec((NC * NS, NP_), lambda i: (0, 0)),
            pl.BlockSpec((_BM, D_OUT), lambda i: (i, 0)),
        ],
        out_specs=pl.BlockSpec((_BM, D_OUT), lambda i: (i, 0)),
        out_shape=jax.ShapeDtypeStruct((NP_, D_OUT), jnp.float32),
    )(agg1, cnt, p)


def kernel(x_user, x_movie, edge_index_um, edge_index_mu,
           Wl0_um, bl0_um, Wr0_um, Wl0_mu, bl0_mu, Wr0_mu,
           Wl1_um, bl1_um, Wr1_um, Wl1_mu, bl1_mu, Wr1_mu):
    f32 = jnp.float32
    s_um = edge_index_um[0].astype(jnp.int32)
    d_um = edge_index_um[1].astype(jnp.int32)
    s_mu = edge_index_mu[0].astype(jnp.int32)
    d_mu = edge_index_mu[1].astype(jnp.int32)
    xu2 = jnp.concatenate([x_user[:, :128], x_user[:, 128:]], axis=0)
    xm2 = jnp.concatenate([x_movie[:, :128], x_movie[:, 128:]], axis=0)
    ones_hbm = jnp.ones((G, 16), f32)
    zf = jnp.zeros((G, 128), f32)
    zc = jnp.zeros((G, 16), f32)

    aggm, aggu = _agg0(xu2, xm2, s_um, d_um, s_mu, d_mu,
                       ones_hbm, zf, zc)
    zcnt = jnp.zeros((NP_ // 16, 16), f32)
    dpad = ((0, 1280 * G - E),)
    d_um_p = jnp.pad(d_um, dpad, constant_values=NP_ - 1)
    d_mu_p = jnp.pad(d_mu, dpad, constant_values=NP_ - 1)
    cm4, cu4 = _degrees(d_um_p, d_mu_p, zcnt)
    cntm = cm4.reshape(NC * NS, NP_)
    cntu = cu4.reshape(NC * NS, NP_)

    xmp = jnp.pad(x_movie, ((0, NP_ - N), (0, 0)))
    xup = jnp.pad(x_user, ((0, NP_ - N), (0, 0)))
    t_m, p_m = _dense0(aggm, cntm, xmp, Wl0_um, bl0_um.reshape(1, -1), Wr0_um,
                       Wl1_mu, Wr1_um, bl1_um.reshape(1, -1))
    t_u, p_u = _dense0(aggu, cntu, xup, Wl0_mu, bl0_mu.reshape(1, -1), Wr0_mu,
                       Wl1_um, Wr1_mu, bl1_mu.reshape(1, -1))

    agg1m, agg1u = _agg1(t_u, t_m, s_um, d_um, s_mu, d_mu, zf)

    out_movie = _combine(agg1m, cntm, p_m)[:N]
    out_user = _combine(agg1u, cntu, p_u)[:N]
    return (out_user, out_movie)
